# baseline (speedup 1.0000x reference)
import functools

import numpy as np
import jax
import jax.numpy as jnp
from jax import lax
from jax.experimental import pallas as pl
from jax.experimental.pallas import tpu as pltpu
from jax.experimental.pallas import tpu_sc as plsc

N = 50000
E = 800000
NODE_IN = 128
EDGE_IN = 4
HID = 64
LAT = 32
NMP = 3

NC = 2
NS = 16
NW = NC * NS

CH = 128
CPW = 200
E_PAD = NW * CPW * CH
EP4 = E_PAD // 4
HCPW = 100
NROW_W = 3136
N_PAD = NS * NROW_W
NP4 = N_PAD // 4
SB = 196
CNTW = 16

EBLK = 2048
NBLK = 1568


@functools.lru_cache(maxsize=None)
def _sc_mesh():
    return plsc.VectorSubcoreMesh(
        core_axis_name="c", subcore_axis_name="s", num_cores=NC, num_subcores=NS
    )


def _silu(v):
    return v * jax.nn.sigmoid(v)


def _gather_body(h_hbm, ridx, cidx, hr_out, hc_out, htab, idx2d,
                 r0, r1, r2, r3, g0, g1, g2, g3, s0, s1, s2, s3):
    cid = lax.axis_index("c")
    sid = lax.axis_index("s")
    wid = sid * NC + cid
    base_c = wid * CPW
    rows = (r0, r1, r2, r3)
    gsem = (g0, g1, g2, g3)
    ssem = (s0, s1, s2, s3)
    tr0 = sid * NROW_W
    pltpu.sync_copy(h_hbm.at[pl.ds(tr0, NROW_W)], htab.at[pl.ds(tr0, NROW_W)])
    plsc.subcore_barrier()
    for idx_hbm, out_hbm in ((ridx, hr_out), (cidx, hc_out)):
        for half in range(CPW // HCPW):
            hbase = base_c + half * HCPW
            pltpu.sync_copy(idx_hbm.at[pl.ds(hbase, HCPW)], idx2d)
            for s in range(4):
                pltpu.async_copy(htab.at[idx2d.at[s]], rows[s], gsem[s])

            def body(t, carry):
                for s in range(4):
                    j = 4 * t + s
                    pltpu.make_async_copy(
                        htab.at[idx2d.at[j]], rows[s], gsem[s]).wait()
                    pltpu.async_copy(
                        rows[s], out_hbm.at[pl.ds((hbase + j) * CH, CH)],
                        ssem[s])
                for s in range(4):
                    j = 4 * t + s
                    pltpu.make_async_copy(
                        rows[s], out_hbm.at[pl.ds((hbase + j) * CH, CH)],
                        ssem[s]).wait()

                    @pl.when(t < HCPW // 4 - 1)
                    def _():
                        pltpu.async_copy(
                            htab.at[idx2d.at[j + 4]], rows[s], gsem[s])
                return carry

            lax.fori_loop(0, HCPW // 4, body, 0)


@functools.lru_cache(maxsize=None)
def _gather_kernel():
    return pl.kernel(
        _gather_body,
        out_type=(
            jax.ShapeDtypeStruct((E_PAD, LAT), jnp.bfloat16),
            jax.ShapeDtypeStruct((E_PAD, LAT), jnp.bfloat16),
        ),
        mesh=_sc_mesh(),
        compiler_params=pltpu.CompilerParams(use_tc_tiling_on_sc=False),
        scratch_types=[
            pltpu.VMEM_SHARED((N_PAD, LAT), jnp.bfloat16),
            pltpu.VMEM((HCPW, CH), jnp.int32),
            pltpu.VMEM((CH, LAT), jnp.bfloat16),
            pltpu.VMEM((CH, LAT), jnp.bfloat16),
            pltpu.VMEM((CH, LAT), jnp.bfloat16),
            pltpu.VMEM((CH, LAT), jnp.bfloat16),
            pltpu.SemaphoreType.DMA,
            pltpu.SemaphoreType.DMA,
            pltpu.SemaphoreType.DMA,
            pltpu.SemaphoreType.DMA,
            pltpu.SemaphoreType.DMA,
            pltpu.SemaphoreType.DMA,
            pltpu.SemaphoreType.DMA,
            pltpu.SemaphoreType.DMA,
        ],
    )


def _scatter_body(vals, sidx, zeros, out, shared, idx2d, vb0, vb1, stage,
                  l0, l1):
    cid = lax.axis_index("c")
    sid = lax.axis_index("s")
    wid = sid * NC + cid
    r0 = sid * NROW_W
    pltpu.sync_copy(zeros.at[pl.ds(r0, NROW_W)], shared.at[pl.ds(r0, NROW_W)])
    plsc.subcore_barrier()

    vbuf = (vb0, vb1)
    lsem = (l0, l1)
    for half in range(CPW // HCPW):
        cbase = wid * CPW + half * HCPW
        pltpu.sync_copy(sidx.at[pl.ds(cbase, HCPW)], idx2d)
        pltpu.async_copy(vals.at[pl.ds(cbase * CH, CH)], vb0, l0)

        def body(t, carry):
            for s in range(2):
                j = 2 * t + s

                @pl.when(j + 1 < HCPW)
                def _():
                    pltpu.async_copy(
                        vals.at[pl.ds((cbase + j + 1) * CH, CH)],
                        vbuf[(s + 1) % 2], lsem[(s + 1) % 2])

                pltpu.make_async_copy(
                    vals.at[pl.ds((cbase + j) * CH, CH)], vbuf[s],
                    lsem[s]).wait()
                pltpu.sync_copy(vbuf[s], shared.at[idx2d.at[j]], add=True)
            return carry

        lax.fori_loop(0, HCPW // 2, body, 0)
    plsc.subcore_barrier()
    for q in range(NROW_W // SB):
        pltpu.sync_copy(shared.at[pl.ds(r0 + q * SB, SB)], stage)
        pltpu.sync_copy(stage, out.at[cid, pl.ds(r0 + q * SB, SB)])


@functools.lru_cache(maxsize=None)
def _scatter_kernel():
    return pl.kernel(
        _scatter_body,
        out_type=jax.ShapeDtypeStruct((NC, N_PAD, LAT), jnp.float32),
        mesh=_sc_mesh(),
        compiler_params=pltpu.CompilerParams(use_tc_tiling_on_sc=False),
        scratch_types=[
            pltpu.VMEM_SHARED((N_PAD, LAT), jnp.float32),
            pltpu.VMEM((HCPW, CH), jnp.int32),
            pltpu.VMEM((CH, LAT), jnp.float32),
            pltpu.VMEM((CH, LAT), jnp.float32),
            pltpu.VMEM((SB, LAT), jnp.float32),
            pltpu.SemaphoreType.DMA,
            pltpu.SemaphoreType.DMA,
        ],
    )


def _count_body(sidx, ones, zeros, out, shared, idx2d, ones_v, stage):
    cid = lax.axis_index("c")
    sid = lax.axis_index("s")
    wid = sid * NC + cid
    r0 = sid * NROW_W
    pltpu.sync_copy(zeros.at[pl.ds(r0, NROW_W)], shared.at[pl.ds(r0, NROW_W)])
    plsc.subcore_barrier()
    pltpu.sync_copy(sidx.at[pl.ds(wid * CPW, CPW)], idx2d)
    pltpu.sync_copy(ones, ones_v)

    def body(j, carry):
        pltpu.sync_copy(ones_v, shared.at[idx2d.at[j]], add=True)
        return carry

    lax.fori_loop(0, CPW, body, 0)
    plsc.subcore_barrier()
    for q in range(NROW_W // SB):
        pltpu.sync_copy(shared.at[pl.ds(r0 + q * SB, SB)], stage)
        pltpu.sync_copy(stage, out.at[cid, pl.ds(r0 + q * SB, SB)])


@functools.lru_cache(maxsize=None)
def _count_kernel():
    return pl.kernel(
        _count_body,
        out_type=jax.ShapeDtypeStruct((NC, N_PAD, CNTW), jnp.float32),
        mesh=_sc_mesh(),
        compiler_params=pltpu.CompilerParams(use_tc_tiling_on_sc=False),
        scratch_types=[
            pltpu.VMEM_SHARED((N_PAD, CNTW), jnp.float32),
            pltpu.VMEM((CPW, CH), jnp.int32),
            pltpu.VMEM((CH, CNTW), jnp.float32),
            pltpu.VMEM((SB, CNTW), jnp.float32),
        ],
    )


def _dot(a, b):
    return jnp.dot(a.astype(jnp.bfloat16), b.astype(jnp.bfloat16),
                   preferred_element_type=jnp.float32)


def _dotg0(a, b):
    return lax.dot_general(a.astype(jnp.bfloat16), b.astype(jnp.bfloat16),
                           (((0,), (0,)), ((), ())),
                           preferred_element_type=jnp.float32)


def _full(shape):
    nd = len(shape)
    return pl.BlockSpec(shape, lambda i: (0,) * nd)


def _kron4(w):
    return jnp.kron(jnp.eye(4, dtype=w.dtype), w)


def _tile4(v):
    return jnp.tile(v, 4).reshape(1, -1)


def _mseg():
    return jnp.kron(jnp.eye(4, dtype=jnp.float32),
                    jnp.full((LAT, LAT), 1.0 / LAT, jnp.float32))


def _ln_packed(h, mseg, g4, b4):
    mu = _dot(h, mseg)
    d = h - mu
    var = _dot(d * d, mseg)
    return d * lax.rsqrt(var + 1e-5) * g4 + b4


def _bsel():
    m = np.zeros((4 * CNTW, 128), np.float32)
    for q in range(4):
        m[CNTW * q, LAT * q:LAT * (q + 1)] = 1.0
    return jnp.asarray(m)


def _node_enc_body(x0, x1, x2, x3, w1_ref, b1_ref, w2_ref, b2_ref,
                   g_ref, beta_ref, o_ref):
    parts = []
    for xq in (x0, x1, x2, x3):
        h = _silu(_dot(xq[...], w1_ref[...]) + b1_ref[...])
        h = _silu(_dot(h, w2_ref[...]) + b2_ref[...])
        mu = jnp.mean(h, axis=-1, keepdims=True)
        d = h - mu
        var = jnp.mean(d * d, axis=-1, keepdims=True)
        parts.append(d * lax.rsqrt(var + 1e-5) * g_ref[...] + beta_ref[...])
    o_ref[...] = jnp.concatenate(parts, axis=1)


def _node_enc_call(xp, p):
    nb = NP4 // NBLK
    xspecs = [
        pl.BlockSpec((NBLK, NODE_IN), lambda i, q=q: (q * nb + i, 0))
        for q in range(4)
    ]
    return pl.pallas_call(
        _node_enc_body,
        grid=(nb,),
        in_specs=xspecs + [
            _full((NODE_IN, HID)), _full((1, HID)),
            _full((HID, LAT)), _full((1, LAT)),
            _full((1, LAT)), _full((1, LAT)),
        ],
        out_specs=pl.BlockSpec((NBLK, 128), lambda i: (i, 0)),
        out_shape=jax.ShapeDtypeStruct((NP4, 128), jnp.float32),
    )(xp, xp, xp, xp,
      p["W1"], p["b1"].reshape(1, -1), p["W2"], p["b2"].reshape(1, -1),
      p["g"].reshape(1, -1), p["beta"].reshape(1, -1))


def _edge_enc_body(x0, x1, x2, x3, w1_ref, b1_ref, w2_ref, b2_ref,
                   g_ref, beta_ref, eye_ref, o_ref):
    parts = []
    for xq in (x0, x1, x2, x3):
        h = _silu(_dotg0(w1_ref[...], xq[...]) + b1_ref[...])
        h = _silu(_dotg0(w2_ref[...], h) + b2_ref[...])
        mu = jnp.mean(h, axis=0, keepdims=True)
        d = h - mu
        var = jnp.mean(d * d, axis=0, keepdims=True)
        ln = d * lax.rsqrt(var + 1e-5) * g_ref[...] + beta_ref[...]
        parts.append(_dotg0(ln, eye_ref[...]))
    o_ref[...] = jnp.concatenate(parts, axis=1)


def _edge_enc_call(ea_t, p):
    nb = EP4 // EBLK
    xspecs = [
        pl.BlockSpec((EDGE_IN, EBLK), lambda i, q=q: (0, q * nb + i))
        for q in range(4)
    ]
    return pl.pallas_call(
        _edge_enc_body,
        grid=(nb,),
        in_specs=xspecs + [
            _full((EDGE_IN, HID)), _full((HID, 1)),
            _full((HID, LAT)), _full((LAT, 1)),
            _full((LAT, 1)), _full((LAT, 1)),
            _full((LAT, LAT)),
        ],
        out_specs=pl.BlockSpec((EBLK, 128), lambda i: (i, 0)),
        out_shape=jax.ShapeDtypeStruct((EP4, 128), jnp.float32),
    )(ea_t, ea_t, ea_t, ea_t,
      p["W1"], p["b1"].reshape(-1, 1), p["W2"], p["b2"].reshape(-1, 1),
      p["g"].reshape(-1, 1), p["beta"].reshape(-1, 1),
      jnp.eye(LAT, dtype=jnp.float32))


def _edge_mlp_body(hr_ref, hc_ref, ea_ref, w1a_ref, w1b_ref, w1c_ref, b1_ref,
                   w2_ref, b2_ref, mseg_ref, g_ref, beta_ref, o_ref):
    z = (_dot(hr_ref[...], w1a_ref[...]) + _dot(hc_ref[...], w1b_ref[...])
         + _dot(ea_ref[...], w1c_ref[...]) + b1_ref[...])
    h = _silu(z)
    h = _silu(_dot(h, w2_ref[...]) + b2_ref[...])
    o_ref[...] = _ln_packed(h, mseg_ref[...], g_ref[...], beta_ref[...])


def _edge_mlp_call(hrp, hcp, eap, p):
    w1 = p["W1"]
    return pl.pallas_call(
        _edge_mlp_body,
        grid=(EP4 // EBLK,),
        in_specs=[
            pl.BlockSpec((EBLK, 128), lambda i: (i, 0)),
            pl.BlockSpec((EBLK, 128), lambda i: (i, 0)),
            pl.BlockSpec((EBLK, 128), lambda i: (i, 0)),
            _full((128, 4 * HID)), _full((128, 4 * HID)), _full((128, 4 * HID)),
            _full((1, 4 * HID)),
            _full((4 * HID, 128)), _full((1, 128)),
            _full((128, 128)), _full((1, 128)), _full((1, 128)),
        ],
        out_specs=pl.BlockSpec((EBLK, 128), lambda i: (i, 0)),
        out_shape=jax.ShapeDtypeStruct((EP4, 128), jnp.float32),
    )(hrp, hcp, eap,
      _kron4(w1[:LAT]), _kron4(w1[LAT:2 * LAT]), _kron4(w1[2 * LAT:]),
      _tile4(p["b1"]), _kron4(p["W2"]), _tile4(p["b2"]),
      _mseg(), _tile4(p["g"]), _tile4(p["beta"]))


def _node_mlp_body(h_ref, s_ref, c_ref, bsel_ref, w1a_ref, w1b_ref, b1_ref,
                   w2_ref, b2_ref, mseg_ref, g_ref, beta_ref, o_ref):
    cnt = _dot(c_ref[0] + c_ref[1], bsel_ref[...])
    aggr = (s_ref[0] + s_ref[1]) * (1.0 / jnp.maximum(cnt, 1.0))
    hcur = h_ref[...]
    z = _dot(hcur, w1a_ref[...]) + _dot(aggr, w1b_ref[...]) + b1_ref[...]
    h = _silu(z)
    h = _silu(_dot(h, w2_ref[...]) + b2_ref[...])
    o_ref[...] = hcur + _ln_packed(h, mseg_ref[...], g_ref[...], beta_ref[...])


def _node_mlp_call(hp, s_parts, c_parts, p):
    w1 = p["W1"]
    return pl.pallas_call(
        _node_mlp_body,
        grid=(NP4 // NBLK,),
        in_specs=[
            pl.BlockSpec((NBLK, 128), lambda i: (i, 0)),
            pl.BlockSpec((NC, NBLK, 128), lambda i: (0, i, 0)),
            pl.BlockSpec((NC, NBLK, 4 * CNTW), lambda i: (0, i, 0)),
            _full((4 * CNTW, 128)),
            _full((128, 4 * HID)), _full((128, 4 * HID)), _full((1, 4 * HID)),
            _full((4 * HID, 128)), _full((1, 128)),
            _full((128, 128)), _full((1, 128)), _full((1, 128)),
        ],
        out_specs=pl.BlockSpec((NBLK, 128), lambda i: (i, 0)),
        out_shape=jax.ShapeDtypeStruct((NP4, 128), jnp.float32),
    )(hp, s_parts, c_parts, _bsel(),
      _kron4(w1[:LAT]), _kron4(w1[LAT:]),
      _tile4(p["b1"]), _kron4(p["W2"]), _tile4(p["b2"]),
      _mseg(), _tile4(p["g"]), _tile4(p["beta"]))


def _dec_body(h_ref, pw1, pb1, pw2, pb2, pw3, uw1, ub1, uw2, ub2, uw3,
              s1_ref, s2_ref, b_ref, o_ref):
    hcur = h_ref[...]
    a = _silu(_dot(hcur, pw1[...]) + pb1[...])
    a = _silu(_dot(a, pw2[...]) + pb2[...])
    outp = _dot(a, pw3[...])
    b = _silu(_dot(hcur, uw1[...]) + ub1[...])
    b = _silu(_dot(b, uw2[...]) + ub2[...])
    outu = _dot(b, uw3[...])
    o_ref[...] = _dot(outp, s1_ref[...]) + _dot(outu, s2_ref[...]) + b_ref[...]


def _dec_call(hp, pp, pu):
    s1 = np.zeros((4, 16), np.float32)
    s2 = np.zeros((12, 16), np.float32)
    for q in range(4):
        s1[q, 4 * q] = 1.0
        for c in range(3):
            s2[3 * q + c, 4 * q + 1 + c] = 1.0
    bcat = jnp.tile(jnp.concatenate([pp["b3"], pu["b3"]]), 4).reshape(1, 16)
    args = [hp]
    specs = [pl.BlockSpec((NBLK, 128), lambda i: (i, 0))]
    for p in (pp, pu):
        for w, b in ((_kron4(p["W1"]), _tile4(p["b1"])),
                     (_kron4(p["W2"]), _tile4(p["b2"]))):
            args += [w, b]
            specs += [_full(w.shape), _full(b.shape)]
        w3 = _kron4(p["W3"])
        args.append(w3)
        specs.append(_full(w3.shape))
    args += [jnp.asarray(s1), jnp.asarray(s2), bcat]
    specs += [_full(s1.shape), _full(s2.shape), _full((1, 16))]
    return pl.pallas_call(
        _dec_body,
        grid=(NP4 // NBLK,),
        in_specs=specs,
        out_specs=pl.BlockSpec((NBLK, 16), lambda i: (i, 0)),
        out_shape=jax.ShapeDtypeStruct((NP4, 16), jnp.float32),
    )(*args)


def kernel(x, edge_index, edge_attr, params):
    row = edge_index[0]
    col = edge_index[1]

    def tau(n):
        return 4 * (n % NP4) + n // NP4

    def eperm(a):
        return a.reshape(4, EP4).transpose(1, 0).reshape(E_PAD)

    pad = E_PAD - E
    gidx_row = eperm(tau(jnp.concatenate(
        [row, jnp.zeros((pad,), jnp.int32)]))).reshape(-1, CH)
    gidx_col = eperm(tau(jnp.concatenate(
        [col, jnp.zeros((pad,), jnp.int32)]))).reshape(-1, CH)
    sidx_col = eperm(tau(jnp.concatenate(
        [col, jnp.full((pad,), N, jnp.int32)]))).reshape(-1, CH)

    zeros32 = jnp.zeros((N_PAD, LAT), jnp.float32)
    zeros16 = jnp.zeros((N_PAD, CNTW), jnp.float32)
    ones16 = jnp.ones((CH, CNTW), jnp.float32)

    xp = jnp.concatenate(
        [x, jnp.zeros((N_PAD - N, NODE_IN), jnp.float32)], axis=0)
    ea_t = jnp.concatenate(
        [edge_attr.T, jnp.zeros((EDGE_IN, pad), jnp.float32)], axis=1)

    hp = _node_enc_call(xp, params["node_enc"])
    eap = _edge_enc_call(ea_t, params["edge_enc"])
    c_parts = _count_kernel()(sidx_col, ones16, zeros16)
    c_parts = c_parts.reshape(NC, NP4, 4 * CNTW)

    for lp in params["mp"]:
        hb = hp.astype(jnp.bfloat16)
        hr, hc = _gather_kernel()(hb.reshape(N_PAD, LAT), gidx_row, gidx_col)
        nep = _edge_mlp_call(hr.reshape(EP4, 128), hc.reshape(EP4, 128),
                             eap, lp["edge_mlp"])
        s_parts = _scatter_kernel()(nep.reshape(E_PAD, LAT), sidx_col, zeros32)
        hp = _node_mlp_call(hp, s_parts.reshape(NC, NP4, 128), c_parts,
                            lp["node_mlp"])
        eap = nep

    out = _dec_call(hp, params["dec_p"], params["dec_U"])
    return out.reshape(NP4, 4, 4).transpose(1, 0, 2).reshape(N_PAD, 4)[:N]


# speedup vs baseline: 1.5307x; 1.5307x over previous
import functools

import numpy as np
import jax
import jax.numpy as jnp
from jax import lax
from jax.experimental import pallas as pl
from jax.experimental.pallas import tpu as pltpu
from jax.experimental.pallas import tpu_sc as plsc

N = 50000
E = 800000
NODE_IN = 128
EDGE_IN = 4
HID = 64
LAT = 32
NMP = 3

NC = 2
NS = 16
NW = NC * NS

CH = 128
CPW = 200
E_PAD = NW * CPW * CH
EP4 = E_PAD // 4
HCPW = 100
NROW_W = 3136
N_PAD = NS * NROW_W
NP4 = N_PAD // 4
SB = 196
CNTW = 16

EBLK = 2048
NBLK = 1568


@functools.lru_cache(maxsize=None)
def _sc_mesh():
    return plsc.VectorSubcoreMesh(
        core_axis_name="c", subcore_axis_name="s", num_cores=NC, num_subcores=NS
    )


def _silu(v):
    return v * jax.nn.sigmoid(v)


def _gather_body(h_hbm, ridx, cidx, hr_out, hc_out, htab, idx2d,
                 r0, r1, r2, r3, g0, g1, g2, g3, s0, s1, s2, s3):
    cid = lax.axis_index("c")
    sid = lax.axis_index("s")
    wid = sid * NC + cid
    base_c = wid * CPW
    rows = (r0, r1, r2, r3)
    gsem = (g0, g1, g2, g3)
    ssem = (s0, s1, s2, s3)
    tr0 = sid * NROW_W
    pltpu.sync_copy(h_hbm.at[pl.ds(tr0, NROW_W)], htab.at[pl.ds(tr0, NROW_W)])
    plsc.subcore_barrier()
    for idx_hbm, out_hbm in ((ridx, hr_out), (cidx, hc_out)):
        for half in range(CPW // HCPW):
            hbase = base_c + half * HCPW
            pltpu.sync_copy(idx_hbm.at[pl.ds(hbase, HCPW)], idx2d)
            for s in range(4):
                pltpu.async_copy(htab.at[idx2d.at[s]], rows[s], gsem[s])

            def body(t, carry):
                for s in range(4):
                    j = 4 * t + s
                    pltpu.make_async_copy(
                        htab.at[idx2d.at[j]], rows[s], gsem[s]).wait()
                    pltpu.async_copy(
                        rows[s], out_hbm.at[pl.ds((hbase + j) * CH, CH)],
                        ssem[s])
                for s in range(4):
                    j = 4 * t + s
                    pltpu.make_async_copy(
                        rows[s], out_hbm.at[pl.ds((hbase + j) * CH, CH)],
                        ssem[s]).wait()

                    @pl.when(t < HCPW // 4 - 1)
                    def _():
                        pltpu.async_copy(
                            htab.at[idx2d.at[j + 4]], rows[s], gsem[s])
                return carry

            lax.fori_loop(0, HCPW // 4, body, 0)


@functools.lru_cache(maxsize=None)
def _gather_kernel():
    return pl.kernel(
        _gather_body,
        out_type=(
            jax.ShapeDtypeStruct((E_PAD, LAT), jnp.float32),
            jax.ShapeDtypeStruct((E_PAD, LAT), jnp.float32),
        ),
        mesh=_sc_mesh(),
        compiler_params=pltpu.CompilerParams(use_tc_tiling_on_sc=False),
        scratch_types=[
            pltpu.VMEM_SHARED((N_PAD, LAT), jnp.float32),
            pltpu.VMEM((HCPW, CH), jnp.int32),
            pltpu.VMEM((CH, LAT), jnp.float32),
            pltpu.VMEM((CH, LAT), jnp.float32),
            pltpu.VMEM((CH, LAT), jnp.float32),
            pltpu.VMEM((CH, LAT), jnp.float32),
            pltpu.SemaphoreType.DMA,
            pltpu.SemaphoreType.DMA,
            pltpu.SemaphoreType.DMA,
            pltpu.SemaphoreType.DMA,
            pltpu.SemaphoreType.DMA,
            pltpu.SemaphoreType.DMA,
            pltpu.SemaphoreType.DMA,
            pltpu.SemaphoreType.DMA,
        ],
    )


def _scatter_body(vals, sidx, zeros, out, shared, idx2d, vb0, vb1, stage,
                  l0, l1):
    cid = lax.axis_index("c")
    sid = lax.axis_index("s")
    wid = sid * NC + cid
    r0 = sid * NROW_W
    pltpu.sync_copy(zeros.at[pl.ds(r0, NROW_W)], shared.at[pl.ds(r0, NROW_W)])
    plsc.subcore_barrier()

    vbuf = (vb0, vb1)
    lsem = (l0, l1)
    for half in range(CPW // HCPW):
        cbase = wid * CPW + half * HCPW
        pltpu.sync_copy(sidx.at[pl.ds(cbase, HCPW)], idx2d)
        pltpu.async_copy(vals.at[pl.ds(cbase * CH, CH)], vb0, l0)

        def body(t, carry):
            for s in range(2):
                j = 2 * t + s

                @pl.when(j + 1 < HCPW)
                def _():
                    pltpu.async_copy(
                        vals.at[pl.ds((cbase + j + 1) * CH, CH)],
                        vbuf[(s + 1) % 2], lsem[(s + 1) % 2])

                pltpu.make_async_copy(
                    vals.at[pl.ds((cbase + j) * CH, CH)], vbuf[s],
                    lsem[s]).wait()
                pltpu.sync_copy(vbuf[s], shared.at[idx2d.at[j]], add=True)
            return carry

        lax.fori_loop(0, HCPW // 2, body, 0)
    plsc.subcore_barrier()
    for q in range(NROW_W // SB):
        pltpu.sync_copy(shared.at[pl.ds(r0 + q * SB, SB)], stage)
        pltpu.sync_copy(stage, out.at[cid, pl.ds(r0 + q * SB, SB)])


@functools.lru_cache(maxsize=None)
def _scatter_kernel():
    return pl.kernel(
        _scatter_body,
        out_type=jax.ShapeDtypeStruct((NC, N_PAD, LAT), jnp.float32),
        mesh=_sc_mesh(),
        compiler_params=pltpu.CompilerParams(use_tc_tiling_on_sc=False),
        scratch_types=[
            pltpu.VMEM_SHARED((N_PAD, LAT), jnp.float32),
            pltpu.VMEM((HCPW, CH), jnp.int32),
            pltpu.VMEM((CH, LAT), jnp.float32),
            pltpu.VMEM((CH, LAT), jnp.float32),
            pltpu.VMEM((SB, LAT), jnp.float32),
            pltpu.SemaphoreType.DMA,
            pltpu.SemaphoreType.DMA,
        ],
    )


def _count_body(sidx, ones, zeros, out, shared, idx2d, ones_v, stage):
    cid = lax.axis_index("c")
    sid = lax.axis_index("s")
    wid = sid * NC + cid
    r0 = sid * NROW_W
    pltpu.sync_copy(zeros.at[pl.ds(r0, NROW_W)], shared.at[pl.ds(r0, NROW_W)])
    plsc.subcore_barrier()
    pltpu.sync_copy(sidx.at[pl.ds(wid * CPW, CPW)], idx2d)
    pltpu.sync_copy(ones, ones_v)

    def body(j, carry):
        pltpu.sync_copy(ones_v, shared.at[idx2d.at[j]], add=True)
        return carry

    lax.fori_loop(0, CPW, body, 0)
    plsc.subcore_barrier()
    for q in range(NROW_W // SB):
        pltpu.sync_copy(shared.at[pl.ds(r0 + q * SB, SB)], stage)
        pltpu.sync_copy(stage, out.at[cid, pl.ds(r0 + q * SB, SB)])


@functools.lru_cache(maxsize=None)
def _count_kernel():
    return pl.kernel(
        _count_body,
        out_type=jax.ShapeDtypeStruct((NC, N_PAD, CNTW), jnp.float32),
        mesh=_sc_mesh(),
        compiler_params=pltpu.CompilerParams(use_tc_tiling_on_sc=False),
        scratch_types=[
            pltpu.VMEM_SHARED((N_PAD, CNTW), jnp.float32),
            pltpu.VMEM((CPW, CH), jnp.int32),
            pltpu.VMEM((CH, CNTW), jnp.float32),
            pltpu.VMEM((SB, CNTW), jnp.float32),
        ],
    )


def _dot(a, b):
    return jnp.dot(a.astype(jnp.bfloat16), b.astype(jnp.bfloat16),
                   preferred_element_type=jnp.float32)


def _dotg0(a, b):
    return lax.dot_general(a.astype(jnp.bfloat16), b.astype(jnp.bfloat16),
                           (((0,), (0,)), ((), ())),
                           preferred_element_type=jnp.float32)


def _full(shape):
    nd = len(shape)
    return pl.BlockSpec(shape, lambda i: (0,) * nd)


def _kron4(w):
    return jnp.kron(jnp.eye(4, dtype=w.dtype), w)


def _tile4(v):
    return jnp.tile(v, 4).reshape(1, -1)


def _mseg():
    return jnp.kron(jnp.eye(4, dtype=jnp.float32),
                    jnp.full((LAT, LAT), 1.0 / LAT, jnp.float32))


def _ln_packed(h, mseg, g4, b4):
    mu = _dot(h, mseg)
    d = h - mu
    var = _dot(d * d, mseg)
    return d * lax.rsqrt(var + 1e-5) * g4 + b4


def _bsel():
    m = np.zeros((4 * CNTW, 128), np.float32)
    for q in range(4):
        m[CNTW * q, LAT * q:LAT * (q + 1)] = 1.0
    return jnp.asarray(m)


def _node_enc_body(x0, x1, x2, x3, w1_ref, b1_ref, w2_ref, b2_ref,
                   g_ref, beta_ref, o_ref):
    parts = []
    for xq in (x0, x1, x2, x3):
        h = _silu(_dot(xq[...], w1_ref[...]) + b1_ref[...])
        h = _silu(_dot(h, w2_ref[...]) + b2_ref[...])
        mu = jnp.mean(h, axis=-1, keepdims=True)
        d = h - mu
        var = jnp.mean(d * d, axis=-1, keepdims=True)
        parts.append(d * lax.rsqrt(var + 1e-5) * g_ref[...] + beta_ref[...])
    o_ref[...] = jnp.concatenate(parts, axis=1)


def _node_enc_call(xp, p):
    nb = NP4 // NBLK
    xspecs = [
        pl.BlockSpec((NBLK, NODE_IN), lambda i, q=q: (q * nb + i, 0))
        for q in range(4)
    ]
    return pl.pallas_call(
        _node_enc_body,
        grid=(nb,),
        in_specs=xspecs + [
            _full((NODE_IN, HID)), _full((1, HID)),
            _full((HID, LAT)), _full((1, LAT)),
            _full((1, LAT)), _full((1, LAT)),
        ],
        out_specs=pl.BlockSpec((NBLK, 128), lambda i: (i, 0)),
        out_shape=jax.ShapeDtypeStruct((NP4, 128), jnp.float32),
    )(xp, xp, xp, xp,
      p["W1"], p["b1"].reshape(1, -1), p["W2"], p["b2"].reshape(1, -1),
      p["g"].reshape(1, -1), p["beta"].reshape(1, -1))


def _edge_enc_body(x0, x1, x2, x3, w1_ref, b1_ref, w2_ref, b2_ref,
                   g_ref, beta_ref, eye_ref, o_ref):
    parts = []
    for xq in (x0, x1, x2, x3):
        h = _silu(_dotg0(w1_ref[...], xq[...]) + b1_ref[...])
        h = _silu(_dotg0(w2_ref[...], h) + b2_ref[...])
        mu = jnp.mean(h, axis=0, keepdims=True)
        d = h - mu
        var = jnp.mean(d * d, axis=0, keepdims=True)
        ln = d * lax.rsqrt(var + 1e-5) * g_ref[...] + beta_ref[...]
        parts.append(_dotg0(ln, eye_ref[...]))
    o_ref[...] = jnp.concatenate(parts, axis=1)


def _edge_enc_call(ea_t, p):
    nb = EP4 // EBLK
    xspecs = [
        pl.BlockSpec((EDGE_IN, EBLK), lambda i, q=q: (0, q * nb + i))
        for q in range(4)
    ]
    return pl.pallas_call(
        _edge_enc_body,
        grid=(nb,),
        in_specs=xspecs + [
            _full((EDGE_IN, HID)), _full((HID, 1)),
            _full((HID, LAT)), _full((LAT, 1)),
            _full((LAT, 1)), _full((LAT, 1)),
            _full((LAT, LAT)),
        ],
        out_specs=pl.BlockSpec((EBLK, 128), lambda i: (i, 0)),
        out_shape=jax.ShapeDtypeStruct((EP4, 128), jnp.float32),
    )(ea_t, ea_t, ea_t, ea_t,
      p["W1"], p["b1"].reshape(-1, 1), p["W2"], p["b2"].reshape(-1, 1),
      p["g"].reshape(-1, 1), p["beta"].reshape(-1, 1),
      jnp.eye(LAT, dtype=jnp.float32))


def _edge_mlp_body(hr_ref, hc_ref, ea_ref, w1a_ref, w1b_ref, w1c_ref, b1_ref,
                   w2_ref, b2_ref, mseg_ref, g_ref, beta_ref, o_ref):
    z = (_dot(hr_ref[...], w1a_ref[...]) + _dot(hc_ref[...], w1b_ref[...])
         + _dot(ea_ref[...], w1c_ref[...]) + b1_ref[...])
    h = _silu(z)
    h = _silu(_dot(h, w2_ref[...]) + b2_ref[...])
    o_ref[...] = _ln_packed(h, mseg_ref[...], g_ref[...], beta_ref[...])


def _edge_mlp_call(hrp, hcp, eap, p):
    w1 = p["W1"]
    return pl.pallas_call(
        _edge_mlp_body,
        grid=(EP4 // EBLK,),
        in_specs=[
            pl.BlockSpec((EBLK, 128), lambda i: (i, 0)),
            pl.BlockSpec((EBLK, 128), lambda i: (i, 0)),
            pl.BlockSpec((EBLK, 128), lambda i: (i, 0)),
            _full((128, 4 * HID)), _full((128, 4 * HID)), _full((128, 4 * HID)),
            _full((1, 4 * HID)),
            _full((4 * HID, 128)), _full((1, 128)),
            _full((128, 128)), _full((1, 128)), _full((1, 128)),
        ],
        out_specs=pl.BlockSpec((EBLK, 128), lambda i: (i, 0)),
        out_shape=jax.ShapeDtypeStruct((EP4, 128), jnp.float32),
    )(hrp, hcp, eap,
      _kron4(w1[:LAT]), _kron4(w1[LAT:2 * LAT]), _kron4(w1[2 * LAT:]),
      _tile4(p["b1"]), _kron4(p["W2"]), _tile4(p["b2"]),
      _mseg(), _tile4(p["g"]), _tile4(p["beta"]))


def _node_mlp_body(h_ref, s_ref, c_ref, bsel_ref, w1a_ref, w1b_ref, b1_ref,
                   w2_ref, b2_ref, mseg_ref, g_ref, beta_ref, o_ref):
    cnt = _dot(c_ref[0] + c_ref[1], bsel_ref[...])
    aggr = (s_ref[0] + s_ref[1]) * (1.0 / jnp.maximum(cnt, 1.0))
    hcur = h_ref[...]
    z = _dot(hcur, w1a_ref[...]) + _dot(aggr, w1b_ref[...]) + b1_ref[...]
    h = _silu(z)
    h = _silu(_dot(h, w2_ref[...]) + b2_ref[...])
    o_ref[...] = hcur + _ln_packed(h, mseg_ref[...], g_ref[...], beta_ref[...])


def _node_mlp_call(hp, s_parts, c_parts, p):
    w1 = p["W1"]
    return pl.pallas_call(
        _node_mlp_body,
        grid=(NP4 // NBLK,),
        in_specs=[
            pl.BlockSpec((NBLK, 128), lambda i: (i, 0)),
            pl.BlockSpec((NC, NBLK, 128), lambda i: (0, i, 0)),
            pl.BlockSpec((NC, NBLK, 4 * CNTW), lambda i: (0, i, 0)),
            _full((4 * CNTW, 128)),
            _full((128, 4 * HID)), _full((128, 4 * HID)), _full((1, 4 * HID)),
            _full((4 * HID, 128)), _full((1, 128)),
            _full((128, 128)), _full((1, 128)), _full((1, 128)),
        ],
        out_specs=pl.BlockSpec((NBLK, 128), lambda i: (i, 0)),
        out_shape=jax.ShapeDtypeStruct((NP4, 128), jnp.float32),
    )(hp, s_parts, c_parts, _bsel(),
      _kron4(w1[:LAT]), _kron4(w1[LAT:]),
      _tile4(p["b1"]), _kron4(p["W2"]), _tile4(p["b2"]),
      _mseg(), _tile4(p["g"]), _tile4(p["beta"]))


def _dec_body(h_ref, pw1, pb1, pw2, pb2, pw3, uw1, ub1, uw2, ub2, uw3,
              s1_ref, s2_ref, b_ref, o_ref):
    hcur = h_ref[...]
    a = _silu(_dot(hcur, pw1[...]) + pb1[...])
    a = _silu(_dot(a, pw2[...]) + pb2[...])
    outp = _dot(a, pw3[...])
    b = _silu(_dot(hcur, uw1[...]) + ub1[...])
    b = _silu(_dot(b, uw2[...]) + ub2[...])
    outu = _dot(b, uw3[...])
    o_ref[...] = _dot(outp, s1_ref[...]) + _dot(outu, s2_ref[...]) + b_ref[...]


def _dec_call(hp, pp, pu):
    s1 = np.zeros((4, 16), np.float32)
    s2 = np.zeros((12, 16), np.float32)
    for q in range(4):
        s1[q, 4 * q] = 1.0
        for c in range(3):
            s2[3 * q + c, 4 * q + 1 + c] = 1.0
    bcat = jnp.tile(jnp.concatenate([pp["b3"], pu["b3"]]), 4).reshape(1, 16)
    args = [hp]
    specs = [pl.BlockSpec((NBLK, 128), lambda i: (i, 0))]
    for p in (pp, pu):
        for w, b in ((_kron4(p["W1"]), _tile4(p["b1"])),
                     (_kron4(p["W2"]), _tile4(p["b2"]))):
            args += [w, b]
            specs += [_full(w.shape), _full(b.shape)]
        w3 = _kron4(p["W3"])
        args.append(w3)
        specs.append(_full(w3.shape))
    args += [jnp.asarray(s1), jnp.asarray(s2), bcat]
    specs += [_full(s1.shape), _full(s2.shape), _full((1, 16))]
    return pl.pallas_call(
        _dec_body,
        grid=(NP4 // NBLK,),
        in_specs=specs,
        out_specs=pl.BlockSpec((NBLK, 16), lambda i: (i, 0)),
        out_shape=jax.ShapeDtypeStruct((NP4, 16), jnp.float32),
    )(*args)


def kernel(x, edge_index, edge_attr, params):
    row = edge_index[0]
    col = edge_index[1]

    def tau(n):
        return 4 * (n % NP4) + n // NP4

    def eperm(a):
        return a.reshape(4, EP4).transpose(1, 0).reshape(E_PAD)

    pad = E_PAD - E
    gidx_row = eperm(tau(jnp.concatenate(
        [row, jnp.zeros((pad,), jnp.int32)]))).reshape(-1, CH)
    gidx_col = eperm(tau(jnp.concatenate(
        [col, jnp.zeros((pad,), jnp.int32)]))).reshape(-1, CH)
    sidx_col = eperm(tau(jnp.concatenate(
        [col, jnp.full((pad,), N, jnp.int32)]))).reshape(-1, CH)

    zeros32 = jnp.zeros((N_PAD, LAT), jnp.float32)
    zeros16 = jnp.zeros((N_PAD, CNTW), jnp.float32)
    ones16 = jnp.ones((CH, CNTW), jnp.float32)

    xp = jnp.concatenate(
        [x, jnp.zeros((N_PAD - N, NODE_IN), jnp.float32)], axis=0)
    ea_t = jnp.concatenate(
        [edge_attr.T, jnp.zeros((EDGE_IN, pad), jnp.float32)], axis=1)

    hp = _node_enc_call(xp, params["node_enc"])
    eap = _edge_enc_call(ea_t, params["edge_enc"])
    c_parts = _count_kernel()(sidx_col, ones16, zeros16)
    c_parts = c_parts.reshape(NC, NP4, 4 * CNTW)

    for lp in params["mp"]:
        hr, hc = _gather_kernel()(hp.reshape(N_PAD, LAT), gidx_row, gidx_col)
        nep = _edge_mlp_call(hr.reshape(EP4, 128), hc.reshape(EP4, 128),
                             eap, lp["edge_mlp"])
        s_parts = _scatter_kernel()(nep.reshape(E_PAD, LAT), sidx_col, zeros32)
        hp = _node_mlp_call(hp, s_parts.reshape(NC, NP4, 128), c_parts,
                            lp["node_mlp"])
        eap = nep

    out = _dec_call(hp, params["dec_p"], params["dec_U"])
    return out.reshape(NP4, 4, 4).transpose(1, 0, 2).reshape(N_PAD, 4)[:N]


# speedup vs baseline: 1.6175x; 1.0567x over previous
import functools

import numpy as np
import jax
import jax.numpy as jnp
from jax import lax
from jax.experimental import pallas as pl
from jax.experimental.pallas import tpu as pltpu
from jax.experimental.pallas import tpu_sc as plsc

N = 50000
E = 800000
NODE_IN = 128
EDGE_IN = 4
HID = 64
LAT = 32
NMP = 3

NC = 2
NS = 16
NW = NC * NS

CH = 128
CPW = 200
E_PAD = NW * CPW * CH
EP4 = E_PAD // 4
HCPW = 100
NROW_W = 3136
N_PAD = NS * NROW_W
NP4 = N_PAD // 4
SB = 196
CNTW = 16

EBLK = 4096
NBLK = 1568


@functools.lru_cache(maxsize=None)
def _sc_mesh():
    return plsc.VectorSubcoreMesh(
        core_axis_name="c", subcore_axis_name="s", num_cores=NC, num_subcores=NS
    )


def _silu(v):
    return v * jax.nn.sigmoid(v)


def _gather_body(h_hbm, ridx, cidx, hr_out, hc_out, htab, idx2d,
                 r0, r1, r2, r3, g0, g1, g2, g3, s0, s1, s2, s3):
    cid = lax.axis_index("c")
    sid = lax.axis_index("s")
    wid = sid * NC + cid
    base_c = wid * CPW
    rows = (r0, r1, r2, r3)
    gsem = (g0, g1, g2, g3)
    ssem = (s0, s1, s2, s3)
    tr0 = sid * NROW_W
    pltpu.sync_copy(h_hbm.at[pl.ds(tr0, NROW_W)], htab.at[pl.ds(tr0, NROW_W)])
    plsc.subcore_barrier()
    for idx_hbm, out_hbm in ((ridx, hr_out), (cidx, hc_out)):
        for half in range(CPW // HCPW):
            hbase = base_c + half * HCPW
            pltpu.sync_copy(idx_hbm.at[pl.ds(hbase, HCPW)], idx2d)
            for s in range(4):
                pltpu.async_copy(htab.at[idx2d.at[s]], rows[s], gsem[s])

            def body(t, carry):
                for s in range(4):
                    j = 4 * t + s
                    pltpu.make_async_copy(
                        htab.at[idx2d.at[j]], rows[s], gsem[s]).wait()
                    pltpu.async_copy(
                        rows[s], out_hbm.at[pl.ds((hbase + j) * CH, CH)],
                        ssem[s])
                for s in range(4):
                    j = 4 * t + s
                    pltpu.make_async_copy(
                        rows[s], out_hbm.at[pl.ds((hbase + j) * CH, CH)],
                        ssem[s]).wait()

                    @pl.when(t < HCPW // 4 - 1)
                    def _():
                        pltpu.async_copy(
                            htab.at[idx2d.at[j + 4]], rows[s], gsem[s])
                return carry

            lax.fori_loop(0, HCPW // 4, body, 0)


@functools.lru_cache(maxsize=None)
def _gather_kernel():
    return pl.kernel(
        _gather_body,
        out_type=(
            jax.ShapeDtypeStruct((E_PAD, LAT), jnp.float32),
            jax.ShapeDtypeStruct((E_PAD, LAT), jnp.float32),
        ),
        mesh=_sc_mesh(),
        compiler_params=pltpu.CompilerParams(use_tc_tiling_on_sc=False),
        scratch_types=[
            pltpu.VMEM_SHARED((N_PAD, LAT), jnp.float32),
            pltpu.VMEM((HCPW, CH), jnp.int32),
            pltpu.VMEM((CH, LAT), jnp.float32),
            pltpu.VMEM((CH, LAT), jnp.float32),
            pltpu.VMEM((CH, LAT), jnp.float32),
            pltpu.VMEM((CH, LAT), jnp.float32),
            pltpu.SemaphoreType.DMA,
            pltpu.SemaphoreType.DMA,
            pltpu.SemaphoreType.DMA,
            pltpu.SemaphoreType.DMA,
            pltpu.SemaphoreType.DMA,
            pltpu.SemaphoreType.DMA,
            pltpu.SemaphoreType.DMA,
            pltpu.SemaphoreType.DMA,
        ],
    )


def _scatter_body(vals, sidx, zeros, out, shared, idx2d, vb0, vb1, stage,
                  l0, l1):
    cid = lax.axis_index("c")
    sid = lax.axis_index("s")
    wid = sid * NC + cid
    r0 = sid * NROW_W
    pltpu.sync_copy(zeros.at[pl.ds(r0, NROW_W)], shared.at[pl.ds(r0, NROW_W)])
    plsc.subcore_barrier()

    vbuf = (vb0, vb1)
    lsem = (l0, l1)
    for half in range(CPW // HCPW):
        cbase = wid * CPW + half * HCPW
        pltpu.sync_copy(sidx.at[pl.ds(cbase, HCPW)], idx2d)
        pltpu.async_copy(vals.at[pl.ds(cbase * CH, CH)], vb0, l0)

        def body(t, carry):
            for s in range(2):
                j = 2 * t + s

                @pl.when(j + 1 < HCPW)
                def _():
                    pltpu.async_copy(
                        vals.at[pl.ds((cbase + j + 1) * CH, CH)],
                        vbuf[(s + 1) % 2], lsem[(s + 1) % 2])

                pltpu.make_async_copy(
                    vals.at[pl.ds((cbase + j) * CH, CH)], vbuf[s],
                    lsem[s]).wait()
                pltpu.sync_copy(vbuf[s], shared.at[idx2d.at[j]], add=True)
            return carry

        lax.fori_loop(0, HCPW // 2, body, 0)
    plsc.subcore_barrier()
    for q in range(NROW_W // SB):
        pltpu.sync_copy(shared.at[pl.ds(r0 + q * SB, SB)], stage)
        pltpu.sync_copy(stage, out.at[cid, pl.ds(r0 + q * SB, SB)])


@functools.lru_cache(maxsize=None)
def _scatter_kernel():
    return pl.kernel(
        _scatter_body,
        out_type=jax.ShapeDtypeStruct((NC, N_PAD, LAT), jnp.float32),
        mesh=_sc_mesh(),
        compiler_params=pltpu.CompilerParams(use_tc_tiling_on_sc=False),
        scratch_types=[
            pltpu.VMEM_SHARED((N_PAD, LAT), jnp.float32),
            pltpu.VMEM((HCPW, CH), jnp.int32),
            pltpu.VMEM((CH, LAT), jnp.float32),
            pltpu.VMEM((CH, LAT), jnp.float32),
            pltpu.VMEM((SB, LAT), jnp.float32),
            pltpu.SemaphoreType.DMA,
            pltpu.SemaphoreType.DMA,
        ],
    )


def _count_body(sidx, ones, zeros, out, shared, idx2d, ones_v, stage):
    cid = lax.axis_index("c")
    sid = lax.axis_index("s")
    wid = sid * NC + cid
    r0 = sid * NROW_W
    pltpu.sync_copy(zeros.at[pl.ds(r0, NROW_W)], shared.at[pl.ds(r0, NROW_W)])
    plsc.subcore_barrier()
    pltpu.sync_copy(sidx.at[pl.ds(wid * CPW, CPW)], idx2d)
    pltpu.sync_copy(ones, ones_v)

    def body(j, carry):
        pltpu.sync_copy(ones_v, shared.at[idx2d.at[j]], add=True)
        return carry

    lax.fori_loop(0, CPW, body, 0)
    plsc.subcore_barrier()
    for q in range(NROW_W // SB):
        pltpu.sync_copy(shared.at[pl.ds(r0 + q * SB, SB)], stage)
        pltpu.sync_copy(stage, out.at[cid, pl.ds(r0 + q * SB, SB)])


@functools.lru_cache(maxsize=None)
def _count_kernel():
    return pl.kernel(
        _count_body,
        out_type=jax.ShapeDtypeStruct((NC, N_PAD, CNTW), jnp.float32),
        mesh=_sc_mesh(),
        compiler_params=pltpu.CompilerParams(use_tc_tiling_on_sc=False),
        scratch_types=[
            pltpu.VMEM_SHARED((N_PAD, CNTW), jnp.float32),
            pltpu.VMEM((CPW, CH), jnp.int32),
            pltpu.VMEM((CH, CNTW), jnp.float32),
            pltpu.VMEM((SB, CNTW), jnp.float32),
        ],
    )


def _dot(a, b):
    return jnp.dot(a.astype(jnp.bfloat16), b.astype(jnp.bfloat16),
                   preferred_element_type=jnp.float32)


def _dotg0(a, b):
    return lax.dot_general(a.astype(jnp.bfloat16), b.astype(jnp.bfloat16),
                           (((0,), (0,)), ((), ())),
                           preferred_element_type=jnp.float32)


def _full(shape):
    nd = len(shape)
    return pl.BlockSpec(shape, lambda i: (0,) * nd)


def _kron4(w):
    return jnp.kron(jnp.eye(4, dtype=w.dtype), w)


def _tile4(v):
    return jnp.tile(v, 4).reshape(1, -1)


def _mseg():
    return jnp.kron(jnp.eye(4, dtype=jnp.float32),
                    jnp.full((LAT, LAT), 1.0 / LAT, jnp.float32))


def _ln_packed(h, mseg, g4, b4):
    mu = _dot(h, mseg)
    d = h - mu
    var = _dot(d * d, mseg)
    return d * lax.rsqrt(var + 1e-5) * g4 + b4


def _bsel():
    m = np.zeros((4 * CNTW, 128), np.float32)
    for q in range(4):
        m[CNTW * q, LAT * q:LAT * (q + 1)] = 1.0
    return jnp.asarray(m)


def _node_enc_body(x0, x1, x2, x3, w1_ref, b1_ref, w2_ref, b2_ref,
                   g_ref, beta_ref, o_ref):
    parts = []
    for xq in (x0, x1, x2, x3):
        h = _silu(_dot(xq[...], w1_ref[...]) + b1_ref[...])
        h = _silu(_dot(h, w2_ref[...]) + b2_ref[...])
        mu = jnp.mean(h, axis=-1, keepdims=True)
        d = h - mu
        var = jnp.mean(d * d, axis=-1, keepdims=True)
        parts.append(d * lax.rsqrt(var + 1e-5) * g_ref[...] + beta_ref[...])
    o_ref[...] = jnp.concatenate(parts, axis=1)


def _node_enc_call(xp, p):
    nb = NP4 // NBLK
    xspecs = [
        pl.BlockSpec((NBLK, NODE_IN), lambda i, q=q: (q * nb + i, 0))
        for q in range(4)
    ]
    return pl.pallas_call(
        _node_enc_body,
        grid=(nb,),
        in_specs=xspecs + [
            _full((NODE_IN, HID)), _full((1, HID)),
            _full((HID, LAT)), _full((1, LAT)),
            _full((1, LAT)), _full((1, LAT)),
        ],
        out_specs=pl.BlockSpec((NBLK, 128), lambda i: (i, 0)),
        out_shape=jax.ShapeDtypeStruct((NP4, 128), jnp.float32),
    )(xp, xp, xp, xp,
      p["W1"], p["b1"].reshape(1, -1), p["W2"], p["b2"].reshape(1, -1),
      p["g"].reshape(1, -1), p["beta"].reshape(1, -1))


def _edge_enc_body(x0, x1, x2, x3, w1_ref, b1_ref, w2_ref, b2_ref,
                   g_ref, beta_ref, eye_ref, o_ref):
    parts = []
    for xq in (x0, x1, x2, x3):
        h = _silu(_dotg0(w1_ref[...], xq[...]) + b1_ref[...])
        h = _silu(_dotg0(w2_ref[...], h) + b2_ref[...])
        mu = jnp.mean(h, axis=0, keepdims=True)
        d = h - mu
        var = jnp.mean(d * d, axis=0, keepdims=True)
        ln = d * lax.rsqrt(var + 1e-5) * g_ref[...] + beta_ref[...]
        parts.append(_dotg0(ln, eye_ref[...]))
    o_ref[...] = jnp.concatenate(parts, axis=1)


def _edge_enc_call(ea_t, p):
    nb = EP4 // EBLK
    xspecs = [
        pl.BlockSpec((EDGE_IN, EBLK), lambda i, q=q: (0, q * nb + i))
        for q in range(4)
    ]
    return pl.pallas_call(
        _edge_enc_body,
        grid=(nb,),
        in_specs=xspecs + [
            _full((EDGE_IN, HID)), _full((HID, 1)),
            _full((HID, LAT)), _full((LAT, 1)),
            _full((LAT, 1)), _full((LAT, 1)),
            _full((LAT, LAT)),
        ],
        out_specs=pl.BlockSpec((EBLK, 128), lambda i: (i, 0)),
        out_shape=jax.ShapeDtypeStruct((EP4, 128), jnp.float32),
    )(ea_t, ea_t, ea_t, ea_t,
      p["W1"], p["b1"].reshape(-1, 1), p["W2"], p["b2"].reshape(-1, 1),
      p["g"].reshape(-1, 1), p["beta"].reshape(-1, 1),
      jnp.eye(LAT, dtype=jnp.float32))


def _edge_mlp_body(hr_ref, hc_ref, ea_ref, w1a_ref, w1b_ref, w1c_ref, b1_ref,
                   w2_ref, b2_ref, mseg_ref, g_ref, beta_ref, o_ref):
    z = (_dot(hr_ref[...], w1a_ref[...]) + _dot(hc_ref[...], w1b_ref[...])
         + _dot(ea_ref[...], w1c_ref[...]) + b1_ref[...])
    h = _silu(z.astype(jnp.bfloat16))
    h = _silu((_dot(h, w2_ref[...]) + b2_ref[...]).astype(jnp.bfloat16))
    o_ref[...] = _ln_packed(h.astype(jnp.float32), mseg_ref[...],
                            g_ref[...], beta_ref[...])


def _edge_mlp_call(hrp, hcp, eap, p):
    w1 = p["W1"]
    return pl.pallas_call(
        _edge_mlp_body,
        grid=(EP4 // EBLK,),
        in_specs=[
            pl.BlockSpec((EBLK, 128), lambda i: (i, 0)),
            pl.BlockSpec((EBLK, 128), lambda i: (i, 0)),
            pl.BlockSpec((EBLK, 128), lambda i: (i, 0)),
            _full((128, 4 * HID)), _full((128, 4 * HID)), _full((128, 4 * HID)),
            _full((1, 4 * HID)),
            _full((4 * HID, 128)), _full((1, 128)),
            _full((128, 128)), _full((1, 128)), _full((1, 128)),
        ],
        out_specs=pl.BlockSpec((EBLK, 128), lambda i: (i, 0)),
        out_shape=jax.ShapeDtypeStruct((EP4, 128), jnp.float32),
    )(hrp, hcp, eap,
      _kron4(w1[:LAT]), _kron4(w1[LAT:2 * LAT]), _kron4(w1[2 * LAT:]),
      _tile4(p["b1"]), _kron4(p["W2"]), _tile4(p["b2"]),
      _mseg(), _tile4(p["g"]), _tile4(p["beta"]))


def _node_mlp_body(h_ref, s_ref, c_ref, bsel_ref, w1a_ref, w1b_ref, b1_ref,
                   w2_ref, b2_ref, mseg_ref, g_ref, beta_ref, o_ref):
    cnt = _dot(c_ref[0] + c_ref[1], bsel_ref[...])
    aggr = (s_ref[0] + s_ref[1]) * (1.0 / jnp.maximum(cnt, 1.0))
    hcur = h_ref[...]
    z = _dot(hcur, w1a_ref[...]) + _dot(aggr, w1b_ref[...]) + b1_ref[...]
    h = _silu(z)
    h = _silu(_dot(h, w2_ref[...]) + b2_ref[...])
    o_ref[...] = hcur + _ln_packed(h, mseg_ref[...], g_ref[...], beta_ref[...])


def _node_mlp_call(hp, s_parts, c_parts, p):
    w1 = p["W1"]
    return pl.pallas_call(
        _node_mlp_body,
        grid=(NP4 // NBLK,),
        in_specs=[
            pl.BlockSpec((NBLK, 128), lambda i: (i, 0)),
            pl.BlockSpec((NC, NBLK, 128), lambda i: (0, i, 0)),
            pl.BlockSpec((NC, NBLK, 4 * CNTW), lambda i: (0, i, 0)),
            _full((4 * CNTW, 128)),
            _full((128, 4 * HID)), _full((128, 4 * HID)), _full((1, 4 * HID)),
            _full((4 * HID, 128)), _full((1, 128)),
            _full((128, 128)), _full((1, 128)), _full((1, 128)),
        ],
        out_specs=pl.BlockSpec((NBLK, 128), lambda i: (i, 0)),
        out_shape=jax.ShapeDtypeStruct((NP4, 128), jnp.float32),
    )(hp, s_parts, c_parts, _bsel(),
      _kron4(w1[:LAT]), _kron4(w1[LAT:]),
      _tile4(p["b1"]), _kron4(p["W2"]), _tile4(p["b2"]),
      _mseg(), _tile4(p["g"]), _tile4(p["beta"]))


def _dec_body(h_ref, pw1, pb1, pw2, pb2, pw3, uw1, ub1, uw2, ub2, uw3,
              s1_ref, s2_ref, b_ref, o_ref):
    hcur = h_ref[...]
    a = _silu(_dot(hcur, pw1[...]) + pb1[...])
    a = _silu(_dot(a, pw2[...]) + pb2[...])
    outp = _dot(a, pw3[...])
    b = _silu(_dot(hcur, uw1[...]) + ub1[...])
    b = _silu(_dot(b, uw2[...]) + ub2[...])
    outu = _dot(b, uw3[...])
    o_ref[...] = _dot(outp, s1_ref[...]) + _dot(outu, s2_ref[...]) + b_ref[...]


def _dec_call(hp, pp, pu):
    s1 = np.zeros((4, 16), np.float32)
    s2 = np.zeros((12, 16), np.float32)
    for q in range(4):
        s1[q, 4 * q] = 1.0
        for c in range(3):
            s2[3 * q + c, 4 * q + 1 + c] = 1.0
    bcat = jnp.tile(jnp.concatenate([pp["b3"], pu["b3"]]), 4).reshape(1, 16)
    args = [hp]
    specs = [pl.BlockSpec((NBLK, 128), lambda i: (i, 0))]
    for p in (pp, pu):
        for w, b in ((_kron4(p["W1"]), _tile4(p["b1"])),
                     (_kron4(p["W2"]), _tile4(p["b2"]))):
            args += [w, b]
            specs += [_full(w.shape), _full(b.shape)]
        w3 = _kron4(p["W3"])
        args.append(w3)
        specs.append(_full(w3.shape))
    args += [jnp.asarray(s1), jnp.asarray(s2), bcat]
    specs += [_full(s1.shape), _full(s2.shape), _full((1, 16))]
    return pl.pallas_call(
        _dec_body,
        grid=(NP4 // NBLK,),
        in_specs=specs,
        out_specs=pl.BlockSpec((NBLK, 16), lambda i: (i, 0)),
        out_shape=jax.ShapeDtypeStruct((NP4, 16), jnp.float32),
    )(*args)


def kernel(x, edge_index, edge_attr, params):
    row = edge_index[0]
    col = edge_index[1]

    def tau(n):
        return 4 * (n % NP4) + n // NP4

    def eperm(a):
        return a.reshape(4, EP4).transpose(1, 0).reshape(E_PAD)

    pad = E_PAD - E
    gidx_row = eperm(tau(jnp.concatenate(
        [row, jnp.zeros((pad,), jnp.int32)]))).reshape(-1, CH)
    gidx_col = eperm(tau(jnp.concatenate(
        [col, jnp.zeros((pad,), jnp.int32)]))).reshape(-1, CH)
    sidx_col = eperm(tau(jnp.concatenate(
        [col, jnp.full((pad,), N, jnp.int32)]))).reshape(-1, CH)

    zeros32 = jnp.zeros((N_PAD, LAT), jnp.float32)
    zeros16 = jnp.zeros((N_PAD, CNTW), jnp.float32)
    ones16 = jnp.ones((CH, CNTW), jnp.float32)

    xp = jnp.concatenate(
        [x, jnp.zeros((N_PAD - N, NODE_IN), jnp.float32)], axis=0)
    ea_t = jnp.concatenate(
        [edge_attr.T, jnp.zeros((EDGE_IN, pad), jnp.float32)], axis=1)

    hp = _node_enc_call(xp, params["node_enc"])
    eap = _edge_enc_call(ea_t, params["edge_enc"])
    c_parts = _count_kernel()(sidx_col, ones16, zeros16)
    c_parts = c_parts.reshape(NC, NP4, 4 * CNTW)

    for lp in params["mp"]:
        hr, hc = _gather_kernel()(hp.reshape(N_PAD, LAT), gidx_row, gidx_col)
        nep = _edge_mlp_call(hr.reshape(EP4, 128), hc.reshape(EP4, 128),
                             eap, lp["edge_mlp"])
        s_parts = _scatter_kernel()(nep.reshape(E_PAD, LAT), sidx_col, zeros32)
        hp = _node_mlp_call(hp, s_parts.reshape(NC, NP4, 128), c_parts,
                            lp["node_mlp"])
        eap = nep

    out = _dec_call(hp, params["dec_p"], params["dec_U"])
    return out.reshape(NP4, 4, 4).transpose(1, 0, 2).reshape(N_PAD, 4)[:N]
